# baseline (speedup 1.0000x reference)
import jax
import jax.numpy as jnp
from jax import lax
from jax.experimental import pallas as pl
from jax.experimental.pallas import tpu as pltpu
from jax.experimental.pallas import tpu_sc as plsc

N_AGENTS = 256
D = 512
GRID = 16
NBLK = 16
OUT = 512
NC = 2
NS = 16
NW = NC * NS
APW = N_AGENTS // NW
ROW = NBLK * N_AGENTS
L = 16


def _sc_p2(o2, out, ov, tab, p2f):
    cid = lax.axis_index("c")
    sid = lax.axis_index("s")
    wid = sid * NC + cid
    base = wid * APW
    pltpu.sync_copy(o2, ov)
    iota = lax.iota(jnp.int32, L)
    ones = jnp.ones((L,), jnp.float32)
    zf = jnp.zeros((L,), jnp.float32)
    neg1 = jnp.full((L,), -1, jnp.int32)

    def agent_body(a, carry):
        i = base + a
        ivec = jnp.full((L,), i, jnp.int32)
        xi = plsc.load_gather(ov, [ivec])
        yi = plsc.load_gather(ov, [ivec + N_AGENTS])
        for k in range(17):
            tab[pl.ds(k * L, L)] = neg1
        for k in range(N_AGENTS // L):
            jvec = iota + (k * L)
            xj = ov[pl.ds(k * L, L)]
            yj = ov[pl.ds(N_AGENTS + k * L, L)]
            relx = (xj - xi) * 2.0 + 8.0
            rely = (yj - yi) * 2.0 + 8.0
            inr = ((relx >= 0.0) & (relx < 16.0)
                   & (rely >= 0.0) & (rely < 16.0))
            valid = inr & (jvec != i)
            cx = relx.astype(jnp.int32)
            cy = rely.astype(jnp.int32)
            oi = jnp.where(valid, cx * GRID + cy, GRID * GRID)
            plsc.store_scatter(tab, [oi], jvec, mask=valid)
            w = plsc.load_gather(tab, [oi])
            m = valid & (w < jvec)

            def fix_body(mc):
                plsc.store_scatter(tab, [oi], jvec, mask=mc)
                w2 = plsc.load_gather(tab, [oi])
                return valid & (w2 < jvec)

            lax.while_loop(lambda mc: jnp.any(mc), fix_body, m)
            for mm in range(16):
                p2f[pl.ds(a * ROW + k * 256 + mm * L, L)] = zf
        for k in range(GRID * GRID // L):
            w = tab[pl.ds(k * L, L)]
            win = w >= 0
            blkv = (iota >> 2) + ((k >> 2) << 2)
            tgt = jnp.where(win, a * ROW + blkv * N_AGENTS + w, 0)
            plsc.store_scatter(p2f, [tgt], ones, mask=win)
        return carry

    lax.fori_loop(0, APW, agent_body, 0)
    pltpu.sync_copy(p2f, out.at[pl.ds(base * ROW, APW * ROW)])


def _b_kernel(h_ref, vg_ref, g_ref):
    h = h_ref[...].astype(jnp.bfloat16)
    g_ref[0] = jnp.dot(h, vg_ref[0],
                       preferred_element_type=jnp.float32).astype(jnp.bfloat16)


def _c_kernel(p2_ref, g_ref, b_ref, o_ref, acc_ref):
    blk = pl.program_id(0)

    @pl.when(blk == 0)
    def _():
        acc_ref[...] = jnp.zeros_like(acc_ref)

    p2 = p2_ref[...].astype(jnp.bfloat16)
    acc_ref[...] += jnp.dot(p2, g_ref[0], preferred_element_type=jnp.float32)

    @pl.when(blk == NBLK - 1)
    def _():
        o_ref[...] = jnp.maximum(acc_ref[...] + b_ref[...], 0.0)


def kernel(hidden_state, obs1, obs2, W, b):
    del obs1
    sc_p2 = pl.kernel(
        _sc_p2,
        out_type=jax.ShapeDtypeStruct((N_AGENTS * ROW,), jnp.float32),
        mesh=plsc.VectorSubcoreMesh(core_axis_name="c", subcore_axis_name="s"),
        compiler_params=pltpu.CompilerParams(needs_layout_passes=False),
        scratch_types=[
            pltpu.VMEM((2 * N_AGENTS,), jnp.float32),
            pltpu.VMEM((272,), jnp.int32),
            pltpu.VMEM((APW * ROW,), jnp.float32),
        ],
    )
    xy = obs2.reshape(N_AGENTS, 2).T.reshape(2 * N_AGENTS)
    p2flat = sc_p2(xy)

    vg = W.reshape(OUT, D, NBLK).transpose(2, 1, 0).astype(jnp.bfloat16)
    g_all = pl.pallas_call(
        _b_kernel,
        grid=(NBLK,),
        in_specs=[
            pl.BlockSpec((N_AGENTS, D), lambda blk: (0, 0)),
            pl.BlockSpec((1, D, OUT), lambda blk: (blk, 0, 0)),
        ],
        out_specs=pl.BlockSpec((1, N_AGENTS, OUT), lambda blk: (blk, 0, 0)),
        out_shape=jax.ShapeDtypeStruct((NBLK, N_AGENTS, OUT), jnp.bfloat16),
    )(hidden_state, vg)

    p2r = p2flat.reshape(N_AGENTS, ROW)
    out = pl.pallas_call(
        _c_kernel,
        grid=(NBLK,),
        in_specs=[
            pl.BlockSpec((N_AGENTS, N_AGENTS), lambda blk: (0, blk)),
            pl.BlockSpec((1, N_AGENTS, OUT), lambda blk: (blk, 0, 0)),
            pl.BlockSpec((1, OUT), lambda blk: (0, 0)),
        ],
        out_specs=pl.BlockSpec((N_AGENTS, OUT), lambda blk: (0, 0)),
        out_shape=jax.ShapeDtypeStruct((N_AGENTS, OUT), jnp.float32),
        scratch_shapes=[pltpu.VMEM((N_AGENTS, OUT), jnp.float32)],
    )(p2r, g_all, b.reshape(1, OUT))
    return out


# speedup vs baseline: 1.2037x; 1.2037x over previous
import jax
import jax.numpy as jnp
from jax import lax
from jax.experimental import pallas as pl
from jax.experimental.pallas import tpu as pltpu
from jax.experimental.pallas import tpu_sc as plsc

N_AGENTS = 256
D = 512
GRID = 16
NBLK = 16
OUT = 512
NC = 2
NS = 16
NW = NC * NS
APW = N_AGENTS // NW
ROW = NBLK * N_AGENTS
L = 16


def _sc_p2(o2, out, ov, tab, p2f):
    cid = lax.axis_index("c")
    sid = lax.axis_index("s")
    wid = sid * NC + cid
    base = wid * APW
    pltpu.sync_copy(o2, ov)
    iota = lax.iota(jnp.int32, L)
    ones = jnp.ones((L,), jnp.float32)
    zf = jnp.zeros((L,), jnp.float32)
    neg1 = jnp.full((L,), -1, jnp.int32)

    def agent_body(a, carry):
        i = base + a
        ivec = jnp.full((L,), i, jnp.int32)
        xi = plsc.load_gather(ov, [ivec])
        yi = plsc.load_gather(ov, [ivec + N_AGENTS])
        for k in range(GRID * GRID // L):
            tab[pl.ds(k * L, L)] = neg1
        for k in range(N_AGENTS // L):
            jvec = iota + (k * L)
            xj = ov[pl.ds(k * L, L)]
            yj = ov[pl.ds(N_AGENTS + k * L, L)]
            relx = (xj - xi) * 2.0 + 8.0
            rely = (yj - yi) * 2.0 + 8.0
            inr = ((relx >= 0.0) & (relx < 16.0)
                   & (rely >= 0.0) & (rely < 16.0))
            valid = inr & (jvec != i)
            cx = relx.astype(jnp.int32)
            cy = rely.astype(jnp.int32)
            oi = jnp.where(valid, cx * GRID + cy, GRID * GRID)
            ks, js = plsc.sort_key_val((oi << 4) | iota, jvec)
            oi_s = ks >> 4
            tab[pl.ds(272, L)] = oi_s
            nxt = plsc.load_gather(tab, [jnp.minimum(iota + 1, 15) + 272])
            win = ((nxt != oi_s) | (iota == 15)) & (oi_s < GRID * GRID)
            plsc.store_scatter(tab, [oi_s], js, mask=win)
            for mm in range(16):
                p2f[pl.ds(a * ROW + k * 256 + mm * L, L)] = zf
        for k in range(GRID * GRID // L):
            w = tab[pl.ds(k * L, L)]
            win = w >= 0
            blkv = (iota >> 2) + ((k >> 2) << 2)
            tgt = jnp.where(win, a * ROW + blkv * N_AGENTS + w, 0)
            plsc.store_scatter(p2f, [tgt], ones, mask=win)
        return carry

    lax.fori_loop(0, APW, agent_body, 0)
    pltpu.sync_copy(p2f, out.at[pl.ds(base * ROW, APW * ROW)])


def _bc_kernel(p2_ref, h_ref, vg_ref, b_ref, o_ref, acc_ref):
    blk = pl.program_id(0)

    @pl.when(blk == 0)
    def _():
        acc_ref[...] = jnp.zeros_like(acc_ref)

    h = h_ref[...].astype(jnp.bfloat16)
    g = jnp.dot(h, vg_ref[0],
                preferred_element_type=jnp.float32).astype(jnp.bfloat16)
    p2 = p2_ref[...].astype(jnp.bfloat16)
    acc_ref[...] += jnp.dot(p2, g, preferred_element_type=jnp.float32)

    @pl.when(blk == NBLK - 1)
    def _():
        o_ref[...] = jnp.maximum(acc_ref[...] + b_ref[...], 0.0)


def kernel(hidden_state, obs1, obs2, W, b):
    del obs1
    sc_p2 = pl.kernel(
        _sc_p2,
        out_type=jax.ShapeDtypeStruct((N_AGENTS * ROW,), jnp.float32),
        mesh=plsc.VectorSubcoreMesh(core_axis_name="c", subcore_axis_name="s"),
        compiler_params=pltpu.CompilerParams(needs_layout_passes=False),
        scratch_types=[
            pltpu.VMEM((2 * N_AGENTS,), jnp.float32),
            pltpu.VMEM((288,), jnp.int32),
            pltpu.VMEM((APW * ROW,), jnp.float32),
        ],
    )
    xy = obs2.reshape(N_AGENTS, 2).T.reshape(2 * N_AGENTS)
    p2flat = sc_p2(xy)

    vg = W.reshape(OUT, D, NBLK).transpose(2, 1, 0).astype(jnp.bfloat16)
    p2r = p2flat.reshape(N_AGENTS, ROW)
    out = pl.pallas_call(
        _bc_kernel,
        grid=(NBLK,),
        in_specs=[
            pl.BlockSpec((N_AGENTS, N_AGENTS), lambda blk: (0, blk)),
            pl.BlockSpec((N_AGENTS, D), lambda blk: (0, 0)),
            pl.BlockSpec((1, D, OUT), lambda blk: (blk, 0, 0)),
            pl.BlockSpec((1, OUT), lambda blk: (0, 0)),
        ],
        out_specs=pl.BlockSpec((N_AGENTS, OUT), lambda blk: (0, 0)),
        out_shape=jax.ShapeDtypeStruct((N_AGENTS, OUT), jnp.float32),
        scratch_shapes=[pltpu.VMEM((N_AGENTS, OUT), jnp.float32)],
    )(p2r, hidden_state, vg, b.reshape(1, OUT))
    return out
